# baseline (speedup 1.0000x reference)
import functools

import jax
import jax.numpy as jnp
import numpy as np
from jax import lax
from jax.experimental import pallas as pl
from jax.experimental.pallas import tpu as pltpu
from jax.experimental.pallas import tpu_sc as plsc

_VOCAB = 100000
_DIM = 64
_B = 4096
_T = 200
_PAD = 0

_NC, _NS, _L = 2, 16, 16
_NW = _NC * _NS
_SEQ_PER_W = _B // _NW
_GROUPS = _SEQ_PER_W // _L

_VB = 10000

_POS_IDS = (10, 12, 13, 14, 43, 44)
_NEG_IDS = (11, 15, 45, 46)
_NBITS = len(_POS_IDS) + len(_NEG_IDS)

_bits_np = np.zeros((64,), dtype=np.int32)
for _k, _v in enumerate(_POS_IDS + _NEG_IDS):
    _bits_np[_v] = 1 << _k


def _tc_head_body(w_ref, b_ref, tab_ref, out_ref):
    i = pl.program_id(0)
    s = jnp.sum(tab_ref[...] * w_ref[...], axis=1, keepdims=True) + b_ref[...]

    @pl.when(i == 0)
    def _():
        row = lax.broadcasted_iota(jnp.int32, (_VB, 1), 0)
        out_ref[...] = jnp.where(row == _PAD, b_ref[...], s)

    @pl.when(i != 0)
    def _():
        out_ref[...] = s


def _head_scores(W, b, table):
    s2d = pl.pallas_call(
        _tc_head_body,
        grid=(_VOCAB // _VB,),
        in_specs=[
            pl.BlockSpec((1, _DIM), lambda i: (0, 0)),
            pl.BlockSpec((1, 1), lambda i: (0, 0)),
            pl.BlockSpec((_VB, _DIM), lambda i: (i, 0)),
        ],
        out_specs=pl.BlockSpec((_VB, 1), lambda i: (i, 0)),
        out_shape=jax.ShapeDtypeStruct((_VOCAB, 1), jnp.float32),
    )(W, b.reshape(1, 1), table)
    return s2d.reshape(_VOCAB)


_sc_mesh = plsc.VectorSubcoreMesh(
    core_axis_name="c", subcore_axis_name="s",
    num_cores=_NC, num_subcores=_NS)


@functools.partial(
    pl.kernel,
    out_type=jax.ShapeDtypeStruct((_B,), jnp.float32),
    mesh=_sc_mesh,
    compiler_params=pltpu.CompilerParams(needs_layout_passes=False),
    scratch_types=[
        pltpu.VMEM((_VOCAB,), jnp.float32),
        pltpu.VMEM((64,), jnp.int32),
        pltpu.VMEM((_SEQ_PER_W * _T,), jnp.int32),
        pltpu.VMEM((_SEQ_PER_W,), jnp.float32),
        pltpu.SemaphoreType.DMA,
        pltpu.SemaphoreType.DMA,
        pltpu.SemaphoreType.DMA,
    ],
)
def _sc_reward(s_hbm, bits_hbm, ids_hbm, out_hbm, s_v, bits_v, ids_v, rew_v,
               sem0, sem1, sem2):
    wid = lax.axis_index("s") * _NC + lax.axis_index("c")
    tok_base = wid * (_SEQ_PER_W * _T)
    c0 = pltpu.async_copy(ids_hbm.at[pl.ds(tok_base, _SEQ_PER_W * _T)], ids_v,
                          sem0)
    c1 = pltpu.async_copy(s_hbm, s_v, sem1)
    c2 = pltpu.async_copy(bits_hbm, bits_v, sem2)
    c0.wait()
    c1.wait()
    c2.wait()

    lanes = lax.iota(jnp.int32, 16)
    inv_t = jnp.full((_L,), 1.0 / _T, dtype=jnp.float32)
    for g in range(_GROUPS):
        base = (g * _L) * _T + lanes * _T

        def body(t, carry, base=base):
            acc, m = carry
            vid = plsc.load_gather(ids_v, [base + t])
            acc = acc + plsc.load_gather(s_v, [vid])
            m = m | plsc.load_gather(bits_v, [jnp.minimum(vid, 63)])
            return acc, m

        acc, m = lax.fori_loop(
            0, _T, body,
            (jnp.zeros((_L,), jnp.float32), jnp.zeros((_L,), jnp.int32)))

        cnt = jnp.zeros((_L,), jnp.int32)
        for k in range(len(_POS_IDS)):
            cnt = cnt + ((m >> k) & 1)
        for k in range(len(_POS_IDS), _NBITS):
            cnt = cnt - ((m >> k) & 1)
        rew_v[pl.ds(g * _L, _L)] = acc * inv_t + 0.5 * cnt.astype(jnp.float32)

    pltpu.sync_copy(rew_v, out_hbm.at[pl.ds(wid * _SEQ_PER_W, _SEQ_PER_W)])


def kernel(input_ids, table, W, b):
    ids = input_ids.reshape(-1).astype(jnp.int32)
    s = _head_scores(W.astype(jnp.float32), b.astype(jnp.float32),
                     table.astype(jnp.float32))
    bits = jnp.asarray(_bits_np)
    return _sc_reward(s, bits, ids)


# speedup vs baseline: 1.2649x; 1.2649x over previous
import functools

import jax
import jax.numpy as jnp
import numpy as np
from jax import lax
from jax.experimental import pallas as pl
from jax.experimental.pallas import tpu as pltpu
from jax.experimental.pallas import tpu_sc as plsc

_VOCAB = 100000
_DIM = 64
_B = 4096
_T = 200
_PAD = 0

_NC, _NS, _L = 2, 16, 16
_NW = _NC * _NS
_SEQ_PER_W = _B // _NW
_GROUPS = _SEQ_PER_W // _L

_SROWS = 784
_TCG = 7
_VB = _SROWS // _TCG * 128

_POS_IDS = (10, 12, 13, 14, 43, 44)
_NEG_IDS = (11, 15, 45, 46)
_NBITS = len(_POS_IDS) + len(_NEG_IDS)

_bits_np = np.zeros((64,), dtype=np.int32)
for _k, _v in enumerate(_POS_IDS + _NEG_IDS):
    _bits_np[_v] = 1 << _k


def _tc_head_body(w_ref, b_ref, tab_ref, out_ref):
    i = pl.program_id(0)
    s = jnp.sum(tab_ref[...] * w_ref[...], axis=1) + b_ref[0, 0]

    @pl.when(i == 0)
    def _():
        row = lax.broadcasted_iota(jnp.int32, (_VB,), 0)
        out_ref[...] = jnp.where(row == _PAD, b_ref[0, 0], s).reshape(
            _VB // 128, 128)

    @pl.when(i != 0)
    def _():
        out_ref[...] = s.reshape(_VB // 128, 128)


def _head_scores(W, b, table):
    return pl.pallas_call(
        _tc_head_body,
        grid=(_TCG,),
        in_specs=[
            pl.BlockSpec((1, _DIM), lambda i: (0, 0)),
            pl.BlockSpec((1, 1), lambda i: (0, 0)),
            pl.BlockSpec((_VB, _DIM), lambda i: (i, 0)),
        ],
        out_specs=pl.BlockSpec((_VB // 128, 128), lambda i: (i, 0)),
        out_shape=jax.ShapeDtypeStruct((_SROWS, 128), jnp.float32),
    )(W, b.reshape(1, 1), table)


_sc_mesh = plsc.VectorSubcoreMesh(
    core_axis_name="c", subcore_axis_name="s",
    num_cores=_NC, num_subcores=_NS)


@functools.partial(
    pl.kernel,
    out_type=jax.ShapeDtypeStruct((_B,), jnp.float32),
    mesh=_sc_mesh,
    compiler_params=pltpu.CompilerParams(needs_layout_passes=False),
    scratch_types=[
        pltpu.VMEM((_SROWS, 128), jnp.float32),
        pltpu.VMEM((64,), jnp.int32),
        pltpu.VMEM((_SEQ_PER_W * _T,), jnp.int32),
        pltpu.VMEM((_SEQ_PER_W,), jnp.float32),
        pltpu.SemaphoreType.DMA,
        pltpu.SemaphoreType.DMA,
        pltpu.SemaphoreType.DMA,
    ],
)
def _sc_reward(s_hbm, bits_hbm, ids_hbm, out_hbm, s_v, bits_v, ids_v, rew_v,
               sem0, sem1, sem2):
    wid = lax.axis_index("s") * _NC + lax.axis_index("c")
    row0 = wid * _SEQ_PER_W
    tok0 = wid * (_SEQ_PER_W * _T)
    c0 = pltpu.async_copy(ids_hbm.at[pl.ds(tok0, _SEQ_PER_W * _T)], ids_v,
                          sem0)
    c1 = pltpu.async_copy(s_hbm, s_v, sem1)
    c2 = pltpu.async_copy(bits_hbm, bits_v, sem2)
    c0.wait()
    c1.wait()
    c2.wait()

    lanes = lax.iota(jnp.int32, 16)
    inv_t = jnp.full((_L,), 1.0 / _T, dtype=jnp.float32)
    for g in range(_GROUPS):
        base = (g * _L) * _T + lanes * _T

        def body(t, carry, base=base):
            acc, m = carry
            vid = plsc.load_gather(ids_v, [base + t])
            acc = acc + plsc.load_gather(s_v, [vid >> 7, vid & 127])
            m = m | plsc.load_gather(bits_v, [jnp.minimum(vid, 63)])
            return acc, m

        acc, m = lax.fori_loop(
            0, _T, body,
            (jnp.zeros((_L,), jnp.float32), jnp.zeros((_L,), jnp.int32)))

        cnt = jnp.zeros((_L,), jnp.int32)
        for k in range(len(_POS_IDS)):
            cnt = cnt + ((m >> k) & 1)
        for k in range(len(_POS_IDS), _NBITS):
            cnt = cnt - ((m >> k) & 1)
        rew_v[pl.ds(g * _L, _L)] = acc * inv_t + 0.5 * cnt.astype(jnp.float32)

    pltpu.sync_copy(rew_v, out_hbm.at[pl.ds(row0, _SEQ_PER_W)])


def kernel(input_ids, table, W, b):
    ids = input_ids.reshape(-1).astype(jnp.int32)
    s2d = _head_scores(W, b, table)
    bits = jnp.asarray(_bits_np)
    return _sc_reward(s2d, bits, ids)
